# Initial kernel scaffold; baseline (speedup 1.0000x reference)
import jax
import jax.numpy as jnp
from jax.experimental import pallas as pl


def kernel(x, edge_index, Wl1, Wr1, att1, bias1, gamma1, beta1, Wl2, Wr2, att2, bias2):
    raise NotImplementedError("write your pallas kernel here")


# speedup vs baseline: 7.0663x; 7.0663x over previous
import functools

import jax
import jax.numpy as jnp
from jax import lax
from jax.experimental import pallas as pl
from jax.experimental.pallas import tpu as pltpu
from jax.experimental.pallas import tpu_sc as plsc

N = 10000
C = 256
CH = 128
E = 320000
ER = E + N
K = 128
EA = 20736
EP = EA * 16
NCHUNK = EA // K
NPAD = 10240
SUPC = 18
SUPE = SUPC * K
CK = 64
SUPCC = SUPE // CK
NBLK = 1000
F32 = jnp.float32

_MESH = plsc.VectorSubcoreMesh(
    core_axis_name="c", subcore_axis_name="s", num_cores=2, num_subcores=16
)


def _att_body(xl_hbm, xr_hbm, src_hbm, dst_hbm, att_hbm, plog_hbm,
              gsrc_v, gdst_v, a_rows, b_rows, att_v, plo,
              sga0, sga1, sgb0, sgb1, spo0, spo1):
  c = lax.axis_index("c")
  s = lax.axis_index("s")
  pltpu.sync_copy(att_hbm.at[pl.ds(c * CH, CH)], att_v)
  attb = [att_v[pl.ds(b * 16, 16)] for b in range(CH // 16)]
  tile_base = s * EA
  row_off = c * N
  sga = [sga0, sga1]
  sgb = [sgb0, sgb1]
  spo = [spo0, spo1]

  def issue(j, slot):
    ia = gsrc_v.at[pl.ds(j * K, K)]
    ib = gdst_v.at[pl.ds(j * K, K)]
    pltpu.async_copy(xl_hbm.at[ia], a_rows.at[slot], sga[slot])
    pltpu.async_copy(xr_hbm.at[ib], b_rows.at[slot], sgb[slot])

  def wait_rows(j, slot):
    ia = gsrc_v.at[pl.ds(j * K, K)]
    ib = gdst_v.at[pl.ds(j * K, K)]
    pltpu.make_async_copy(xl_hbm.at[ia], a_rows.at[slot], sga[slot]).wait()
    pltpu.make_async_copy(xr_hbm.at[ib], b_rows.at[slot], sgb[slot]).wait()

  def po_ref(sbase, j, slot):
    return (plo.at[slot],
            plog_hbm.at[pl.ds(c * EP + sbase + j * K, K)])

  def super_loop(u, carry):
    sbase = tile_base + u * SUPE

    @pl.when(u > 0)
    def _():
      for slot in range(2):
        src_r, dst_r = po_ref(sbase, slot - 2, slot)
        pltpu.make_async_copy(src_r, dst_r, spo[slot]).wait()

    pltpu.sync_copy(src_hbm.at[pl.ds(sbase, SUPE)], gsrc_v)
    pltpu.sync_copy(dst_hbm.at[pl.ds(sbase, SUPE)], gdst_v)

    def off(r, cy):
      sl = pl.ds(r * 16, 16)
      gsrc_v[sl] = gsrc_v[sl] + row_off
      gdst_v[sl] = gdst_v[sl] + row_off
      return cy

    lax.fori_loop(0, SUPE // 16, off, 0)
    issue(0, 0)

    def pair(i2, cy):
      for b in range(2):
        k = 2 * i2 + b
        slot = b
        if b == 0:
          issue(k + 1, 1)
        else:
          @pl.when(i2 < SUPC // 2 - 1)
          def _():
            issue(k + 1, 0)
        wait_rows(k, slot)

        @pl.when(i2 > 0)
        def _():
          src_r, dst_r = po_ref(sbase, k, slot)
          pltpu.make_async_copy(src_r, dst_r, spo[slot]).wait()

        def edge(j2, cy2):
          for e in range(2):
            j = j2 * 2 + e
            acc = jnp.zeros((16,), F32)
            for bb in range(CH // 16):
              sl = pl.ds(bb * 16, 16)
              z = a_rows[slot, j, sl] + b_rows[slot, j, sl]
              lr = jnp.maximum(z, 0.2 * z)
              acc = acc + attb[bb] * lr
            plo[slot, j, pl.ds(0, 16)] = acc
          return cy2

        lax.fori_loop(0, K // 2, edge, 0)
        src_r, dst_r = po_ref(sbase, k, slot)
        pltpu.async_copy(src_r, dst_r, spo[slot])
      return cy

    lax.fori_loop(0, SUPC // 2, pair, 0)
    return carry

  lax.fori_loop(0, NCHUNK // SUPC, super_loop, 0)
  for slot in range(2):
    src_r, dst_r = po_ref(tile_base, SUPC - 2 + slot, slot)
    pltpu.make_async_copy(src_r, dst_r, spo[slot]).wait()


_att = pl.kernel(
    _att_body,
    out_type=jax.ShapeDtypeStruct((2 * EP, 16), F32),
    mesh=_MESH,
    scratch_types=[
        pltpu.VMEM((SUPE,), jnp.int32),
        pltpu.VMEM((SUPE,), jnp.int32),
        pltpu.VMEM((2, K, CH), F32),
        pltpu.VMEM((2, K, CH), F32),
        pltpu.VMEM((CH,), F32),
        pltpu.VMEM((2, K, 16), F32),
        pltpu.SemaphoreType.DMA,
        pltpu.SemaphoreType.DMA,
        pltpu.SemaphoreType.DMA,
        pltpu.SemaphoreType.DMA,
        pltpu.SemaphoreType.DMA,
        pltpu.SemaphoreType.DMA,
    ],
)


EVB = 4096


def _evred_body(p0_ref, p1_ref, ev_ref):
  i = pl.program_id(0)
  a = p0_ref[...] + p1_ref[...]
  s = jnp.sum(a, axis=1)
  rr = EVB // 128
  ids = (i * EVB
         + lax.broadcasted_iota(jnp.int32, (rr, 128), 0) * 128
         + lax.broadcasted_iota(jnp.int32, (rr, 128), 1))
  ev = jnp.exp(s).reshape(rr, 128)
  ev_ref[...] = jnp.where(ids < ER, ev, 0.0)


_evred = pl.pallas_call(
    _evred_body,
    grid=(EP // EVB,),
    in_specs=[
        pl.BlockSpec((EVB, 16), lambda i: (i, 0)),
        pl.BlockSpec((EVB, 16), lambda i: (EP // EVB + i, 0)),
    ],
    out_specs=pl.BlockSpec((EVB // 128, 128), lambda i: (i, 0)),
    out_shape=jax.ShapeDtypeStruct((EP // 128, 128), F32),
)


CK = 64
CSUP = 768
CSUPC = CSUP // CK


def _msg_body(xl_hbm, src_hbm, dst_hbm, ev_hbm, z16_hbm, out_hbm, den_hbm,
              gsrc_v, dstb_v, evb_v, di_v, rows, zb1, u_sh, den_sh,
              sg0, sg1):
  c = lax.axis_index("c")
  s = lax.axis_index("s")
  row_off = c * N
  tile_base = s * EA
  sg = [sg0, sg1]

  zb1[...] = jnp.zeros((16,), F32)
  for kk in range(NPAD // (16 * 16)):
    br = 640 * s + 16 * kk
    pltpu.sync_copy(zb1, den_sh.at[pl.ds(br, 16)])

    @pl.when(br < N)
    def _():
      pltpu.sync_copy(z16_hbm, u_sh.at[pl.ds(br, 16)])

  plsc.subcore_barrier()

  def g_refs(j, slot):
    return (xl_hbm.at[gsrc_v.at[pl.ds(j * CK, CK)]], rows.at[slot], sg[slot])

  def super_loop(u, carry):
    sbase = tile_base + u * CSUP
    pltpu.sync_copy(src_hbm.at[pl.ds(sbase, CSUP)], gsrc_v)
    pltpu.sync_copy(dst_hbm.at[pl.ds(sbase, CSUP)], dstb_v)
    pltpu.sync_copy(ev_hbm.at[pl.ds(sbase, CSUP)], evb_v.at[pl.ds(0, CSUP)])

    def off(r, cy):
      sl = pl.ds(r * 16, 16)
      gsrc_v[sl] = gsrc_v[sl] + row_off
      return cy

    lax.fori_loop(0, CSUP // 16, off, 0)
    pltpu.async_copy(*g_refs(0, 0))

    def pair(t, cy):
      for b in range(2):
        j = 2 * t + b
        slot = b
        if b == 0:
          pltpu.async_copy(*g_refs(j + 1, 1))
        else:
          @pl.when(t < CSUPC // 2 - 1)
          def _():
            pltpu.async_copy(*g_refs(j + 1, 0))
        pltpu.make_async_copy(*g_refs(j, slot)).wait()
        for g in range(CK // 16):
          di_v[pl.ds(g * 16, 16)] = dstb_v[pl.ds(j * CK + g * 16, 16)]

        def scale(j2, cy2):
          for e in range(2):
            jj = j2 * 2 + e
            w = evb_v[pl.ds(j * CK + jj, 16)][0]
            for bb in range(8):
              rows[slot, jj, pl.ds(bb * 16, 16)] = (
                  rows[slot, jj, pl.ds(bb * 16, 16)] * w)
          return cy2

        lax.fori_loop(0, CK // 2, scale, 0)
        pltpu.sync_copy(rows.at[slot], u_sh.at[di_v], add=True)
        pltpu.sync_copy(evb_v.at[pl.ds(j * CK, CK)], den_sh.at[di_v], add=True)
      return cy

    lax.fori_loop(0, CSUPC // 2, pair, 0)
    return carry

  lax.fori_loop(0, EA // CSUP, super_loop, 0)
  plsc.subcore_barrier()

  pltpu.sync_copy(den_sh.at[pl.ds(640 * s, 640)],
                  den_hbm.at[pl.ds(c * NPAD + 640 * s, 640)])

  @pl.when(s < 15)
  def _():
    pltpu.sync_copy(u_sh.at[pl.ds(640 * s, 640)],
                    out_hbm.at[pl.ds(row_off + 640 * s, 640)])

  @pl.when(s == 15)
  def _():
    pltpu.sync_copy(u_sh.at[pl.ds(9600, 400)],
                    out_hbm.at[pl.ds(row_off + 9600, 400)])


_msg = pl.kernel(
    _msg_body,
    out_type=[jax.ShapeDtypeStruct((2 * N, CH), F32),
              jax.ShapeDtypeStruct((2 * NPAD,), F32)],
    mesh=_MESH,
    scratch_types=[
        pltpu.VMEM((CSUP,), jnp.int32),
        pltpu.VMEM((CSUP,), jnp.int32),
        pltpu.VMEM((CSUP + 16,), F32),
        pltpu.VMEM((CK,), jnp.int32),
        pltpu.VMEM((2, CK, CH), F32),
        pltpu.VMEM((16,), F32),
        pltpu.VMEM_SHARED((NPAD, CH), F32),
        pltpu.VMEM_SHARED((NPAD,), F32),
        pltpu.SemaphoreType.DMA,
        pltpu.SemaphoreType.DMA,
    ],
)


_DOT = functools.partial(
    jnp.dot, preferred_element_type=F32, precision=lax.Precision.HIGHEST)


def _pre_body(x_ref, wl_ref, wr_ref, xl_ref, xr_ref):
  xb = x_ref[...]
  xl = _DOT(xb, wl_ref[...])
  xr = _DOT(xb, wr_ref[...])
  xl_ref[0] = xl[:, :CH]
  xl_ref[1] = xl[:, CH:]
  xr_ref[0] = xr[:, :CH]
  xr_ref[1] = xr[:, CH:]


_pre = pl.pallas_call(
    _pre_body,
    grid=(N // NBLK,),
    in_specs=[
        pl.BlockSpec((NBLK, 128), lambda i: (i, 0)),
        pl.BlockSpec((128, C), lambda i: (0, 0)),
        pl.BlockSpec((128, C), lambda i: (0, 0)),
    ],
    out_specs=[pl.BlockSpec((2, NBLK, CH), lambda i: (0, i, 0))] * 2,
    out_shape=[jax.ShapeDtypeStruct((2, N, CH), F32)] * 2,
)


def _unorm(u_ref, den_ref):
  u = u_ref[...]
  den = den_ref[...]
  o0 = u[0] / (den[:, 0:1] + 1e-16)
  o1 = u[1] / (den[:, 1:2] + 1e-16)
  return jnp.concatenate([o0[None], o1[None]], axis=0)


def _stats_body(u_ref, den_ref, st_ref):
  i = pl.program_id(0)
  u = _unorm(u_ref, den_ref)
  sm = jnp.sum(u, axis=1)
  sq = jnp.sum(u * u, axis=1)
  blk = jnp.concatenate(
      [sm[:, None, :], sq[:, None, :], jnp.zeros((2, 6, CH), F32)], axis=1)

  @pl.when(i == 0)
  def _():
    st_ref[...] = blk

  @pl.when(i != 0)
  def _():
    st_ref[...] = st_ref[...] + blk


_stats = pl.pallas_call(
    _stats_body,
    grid=(N // NBLK,),
    in_specs=[
        pl.BlockSpec((2, NBLK, CH), lambda i: (0, i, 0)),
        pl.BlockSpec((NBLK, 2), lambda i: (i, 0)),
    ],
    out_specs=pl.BlockSpec((2, 8, CH), lambda i: (0, 0, 0)),
    out_shape=jax.ShapeDtypeStruct((2, 8, CH), F32),
)


def _mid_body(u_ref, den_ref, st_ref, g_ref, b_ref, wl_ref, wr_ref,
              xl_ref, xr_ref):
  u = _unorm(u_ref, den_ref)
  h = jnp.concatenate([u[0], u[1]], axis=1)
  st = st_ref[...]
  sm = jnp.concatenate([st[0, 0], st[1, 0]])[None]
  sq = jnp.concatenate([st[0, 1], st[1, 1]])[None]
  mu = sm / N
  var = sq / N - mu * mu
  hn = (h - mu) * lax.rsqrt(var + 1e-5) * g_ref[...] + b_ref[...]
  hr = jnp.maximum(hn, 0.0)
  xl = _DOT(hr, wl_ref[...])
  xr = _DOT(hr, wr_ref[...])
  xl_ref[0] = xl[:, :CH]
  xl_ref[1] = xl[:, CH:]
  xr_ref[0] = xr[:, :CH]
  xr_ref[1] = xr[:, CH:]


_mid = pl.pallas_call(
    _mid_body,
    grid=(N // NBLK,),
    in_specs=[
        pl.BlockSpec((2, NBLK, CH), lambda i: (0, i, 0)),
        pl.BlockSpec((NBLK, 2), lambda i: (i, 0)),
        pl.BlockSpec((2, 8, CH), lambda i: (0, 0, 0)),
        pl.BlockSpec((1, C), lambda i: (0, 0)),
        pl.BlockSpec((1, C), lambda i: (0, 0)),
        pl.BlockSpec((C, C), lambda i: (0, 0)),
        pl.BlockSpec((C, C), lambda i: (0, 0)),
    ],
    out_specs=[pl.BlockSpec((2, NBLK, CH), lambda i: (0, i, 0))] * 2,
    out_shape=[jax.ShapeDtypeStruct((2, N, CH), F32)] * 2,
)


def _post_body(u_ref, den_ref, st_ref, b2_ref, mp_ref, dp_ref):
  u = _unorm(u_ref, den_ref)
  st = st_ref[...]
  mu = (st[0, 0] / N)[None]
  var = (st[0, 1] / N)[None] - mu * mu
  mp_ref[...] = (u[0] - mu) * lax.rsqrt(var + 1e-5)
  xd = u[1] + b2_ref[...][:, CH:]
  dp_ref[...] = jnp.maximum(xd, 0.0) + jnp.log1p(jnp.exp(-jnp.abs(xd)))


_post = pl.pallas_call(
    _post_body,
    grid=(N // NBLK,),
    in_specs=[
        pl.BlockSpec((2, NBLK, CH), lambda i: (0, i, 0)),
        pl.BlockSpec((NBLK, 2), lambda i: (i, 0)),
        pl.BlockSpec((2, 8, CH), lambda i: (0, 0, 0)),
        pl.BlockSpec((1, C), lambda i: (0, 0)),
    ],
    out_specs=[pl.BlockSpec((NBLK, CH), lambda i: (i, 0))] * 2,
    out_shape=[jax.ShapeDtypeStruct((N, CH), F32)] * 2,
)


def kernel(x, edge_index, Wl1, Wr1, att1, bias1, gamma1, beta1,
           Wl2, Wr2, att2, bias2):
  del bias1
  src = edge_index[0].astype(jnp.int32)
  dst = edge_index[1].astype(jnp.int32)
  loops = jnp.arange(N, dtype=jnp.int32)
  pad = jnp.zeros((EP - ER,), jnp.int32)
  srcp = jnp.concatenate([src, loops, pad])
  dstp = jnp.concatenate([dst, loops, pad])
  z16 = jnp.zeros((16, CH), F32)

  xl1, xr1 = _pre(x, Wl1, Wr1)
  xl1f = xl1.reshape(2 * N, CH)
  plog1 = _att(xl1f, xr1.reshape(2 * N, CH), srcp, dstp, att1.reshape(C))
  ev1 = _evred(plog1, plog1).reshape(EP)
  u1, den1 = _msg(xl1f, srcp, dstp, ev1, z16)
  u1c = u1.reshape(2, N, CH)
  den1t = den1.reshape(2, NPAD)[:, :N].T
  st1 = _stats(u1c, den1t)
  xl2, xr2 = _mid(u1c, den1t, st1, gamma1.reshape(1, C), beta1.reshape(1, C),
                  Wl2, Wr2)
  xl2f = xl2.reshape(2 * N, CH)
  plog2 = _att(xl2f, xr2.reshape(2 * N, CH), srcp, dstp, att2.reshape(C))
  ev2 = _evred(plog2, plog2).reshape(EP)
  u2, den2 = _msg(xl2f, srcp, dstp, ev2, z16)
  u2c = u2.reshape(2, N, CH)
  den2t = den2.reshape(2, NPAD)[:, :N].T
  st2 = _stats(u2c, den2t)
  mean_part, disp_part = _post(u2c, den2t, st2, bias2.reshape(1, C))
  return (mean_part, disp_part)


# speedup vs baseline: 7.1681x; 1.0144x over previous
import functools

import jax
import jax.numpy as jnp
from jax import lax
from jax.experimental import pallas as pl
from jax.experimental.pallas import tpu as pltpu
from jax.experimental.pallas import tpu_sc as plsc

N = 10000
C = 256
CH = 128
E = 320000
ER = E + N
K = 128
EA = 20736
EP = EA * 16
NCHUNK = EA // K
NPAD = 10240
SUPC = 54
SUPE = SUPC * K
CK = 64
SUPCC = SUPE // CK
NBLK = 1000
F32 = jnp.float32

_MESH = plsc.VectorSubcoreMesh(
    core_axis_name="c", subcore_axis_name="s", num_cores=2, num_subcores=16
)


def _att_body(xl_hbm, xr_hbm, src_hbm, dst_hbm, att_hbm, plog_hbm,
              gsrc_v, gdst_v, a_rows, b_rows, att_v, plo,
              sga0, sga1, sgb0, sgb1, spo0, spo1):
  c = lax.axis_index("c")
  s = lax.axis_index("s")
  pltpu.sync_copy(att_hbm.at[pl.ds(c * CH, CH)], att_v)
  attb = [att_v[pl.ds(b * 16, 16)] for b in range(CH // 16)]
  tile_base = s * EA
  row_off = c * N
  sga = [sga0, sga1]
  sgb = [sgb0, sgb1]
  spo = [spo0, spo1]

  def issue(j, slot):
    ia = gsrc_v.at[pl.ds(j * K, K)]
    ib = gdst_v.at[pl.ds(j * K, K)]
    pltpu.async_copy(xl_hbm.at[ia], a_rows.at[slot], sga[slot])
    pltpu.async_copy(xr_hbm.at[ib], b_rows.at[slot], sgb[slot])

  def wait_rows(j, slot):
    ia = gsrc_v.at[pl.ds(j * K, K)]
    ib = gdst_v.at[pl.ds(j * K, K)]
    pltpu.make_async_copy(xl_hbm.at[ia], a_rows.at[slot], sga[slot]).wait()
    pltpu.make_async_copy(xr_hbm.at[ib], b_rows.at[slot], sgb[slot]).wait()

  def po_ref(sbase, j, slot):
    return (plo.at[slot],
            plog_hbm.at[pl.ds(c * EP + sbase + j * K, K)])

  def super_loop(u, carry):
    sbase = tile_base + u * SUPE

    @pl.when(u > 0)
    def _():
      for slot in range(2):
        src_r, dst_r = po_ref(sbase, slot - 2, slot)
        pltpu.make_async_copy(src_r, dst_r, spo[slot]).wait()

    pltpu.sync_copy(src_hbm.at[pl.ds(sbase, SUPE)], gsrc_v)
    pltpu.sync_copy(dst_hbm.at[pl.ds(sbase, SUPE)], gdst_v)

    def off(r, cy):
      sl = pl.ds(r * 16, 16)
      gsrc_v[sl] = gsrc_v[sl] + row_off
      gdst_v[sl] = gdst_v[sl] + row_off
      return cy

    lax.fori_loop(0, SUPE // 16, off, 0)
    issue(0, 0)

    def pair(i2, cy):
      for b in range(2):
        k = 2 * i2 + b
        slot = b
        if b == 0:
          issue(k + 1, 1)
        else:
          @pl.when(i2 < SUPC // 2 - 1)
          def _():
            issue(k + 1, 0)
        wait_rows(k, slot)

        @pl.when(i2 > 0)
        def _():
          src_r, dst_r = po_ref(sbase, k, slot)
          pltpu.make_async_copy(src_r, dst_r, spo[slot]).wait()

        def edge(j2, cy2):
          for e in range(2):
            j = j2 * 2 + e
            acc = jnp.zeros((16,), F32)
            for bb in range(CH // 16):
              sl = pl.ds(bb * 16, 16)
              z = a_rows[slot, j, sl] + b_rows[slot, j, sl]
              lr = jnp.maximum(z, 0.2 * z)
              acc = acc + attb[bb] * lr
            plo[slot, j, pl.ds(0, 16)] = acc
          return cy2

        lax.fori_loop(0, K // 2, edge, 0)
        src_r, dst_r = po_ref(sbase, k, slot)
        pltpu.async_copy(src_r, dst_r, spo[slot])
      return cy

    lax.fori_loop(0, SUPC // 2, pair, 0)
    return carry

  lax.fori_loop(0, NCHUNK // SUPC, super_loop, 0)
  for slot in range(2):
    src_r, dst_r = po_ref(tile_base, SUPC - 2 + slot, slot)
    pltpu.make_async_copy(src_r, dst_r, spo[slot]).wait()


_att = pl.kernel(
    _att_body,
    out_type=jax.ShapeDtypeStruct((2 * EP, 16), F32),
    mesh=_MESH,
    scratch_types=[
        pltpu.VMEM((SUPE,), jnp.int32),
        pltpu.VMEM((SUPE,), jnp.int32),
        pltpu.VMEM((2, K, CH), F32),
        pltpu.VMEM((2, K, CH), F32),
        pltpu.VMEM((CH,), F32),
        pltpu.VMEM((2, K, 16), F32),
        pltpu.SemaphoreType.DMA,
        pltpu.SemaphoreType.DMA,
        pltpu.SemaphoreType.DMA,
        pltpu.SemaphoreType.DMA,
        pltpu.SemaphoreType.DMA,
        pltpu.SemaphoreType.DMA,
    ],
)


EVB = 4096


def _evred_body(p0_ref, p1_ref, ev_ref):
  i = pl.program_id(0)
  a = p0_ref[...] + p1_ref[...]
  s = jnp.sum(a, axis=1)
  rr = EVB // 128
  ids = (i * EVB
         + lax.broadcasted_iota(jnp.int32, (rr, 128), 0) * 128
         + lax.broadcasted_iota(jnp.int32, (rr, 128), 1))
  ev = jnp.exp(s).reshape(rr, 128)
  ev_ref[...] = jnp.where(ids < ER, ev, 0.0)


_evred = pl.pallas_call(
    _evred_body,
    grid=(EP // EVB,),
    in_specs=[
        pl.BlockSpec((EVB, 16), lambda i: (i, 0)),
        pl.BlockSpec((EVB, 16), lambda i: (EP // EVB + i, 0)),
    ],
    out_specs=pl.BlockSpec((EVB // 128, 128), lambda i: (i, 0)),
    out_shape=jax.ShapeDtypeStruct((EP // 128, 128), F32),
)


CK = 64
CSUP = 768
CSUPC = CSUP // CK


def _msg_body(xl_hbm, src_hbm, dst_hbm, ev_hbm, z16_hbm, out_hbm, den_hbm,
              gsrc_v, dstb_v, evb_v, di_v, rows, zb1, u_sh, den_sh,
              sg0, sg1):
  c = lax.axis_index("c")
  s = lax.axis_index("s")
  row_off = c * N
  tile_base = s * EA
  sg = [sg0, sg1]

  zb1[...] = jnp.zeros((16,), F32)
  for kk in range(NPAD // (16 * 16)):
    br = 640 * s + 16 * kk
    pltpu.sync_copy(zb1, den_sh.at[pl.ds(br, 16)])

    @pl.when(br < N)
    def _():
      pltpu.sync_copy(z16_hbm, u_sh.at[pl.ds(br, 16)])

  plsc.subcore_barrier()

  def g_refs(j, slot):
    return (xl_hbm.at[gsrc_v.at[pl.ds(j * CK, CK)]], rows.at[slot], sg[slot])

  def super_loop(u, carry):
    sbase = tile_base + u * CSUP
    pltpu.sync_copy(src_hbm.at[pl.ds(sbase, CSUP)], gsrc_v)
    pltpu.sync_copy(dst_hbm.at[pl.ds(sbase, CSUP)], dstb_v)
    pltpu.sync_copy(ev_hbm.at[pl.ds(sbase, CSUP)], evb_v.at[pl.ds(0, CSUP)])

    def off(r, cy):
      sl = pl.ds(r * 16, 16)
      gsrc_v[sl] = gsrc_v[sl] + row_off
      return cy

    lax.fori_loop(0, CSUP // 16, off, 0)
    pltpu.async_copy(*g_refs(0, 0))

    def pair(t, cy):
      for b in range(2):
        j = 2 * t + b
        slot = b
        if b == 0:
          pltpu.async_copy(*g_refs(j + 1, 1))
        else:
          @pl.when(t < CSUPC // 2 - 1)
          def _():
            pltpu.async_copy(*g_refs(j + 1, 0))
        pltpu.make_async_copy(*g_refs(j, slot)).wait()
        for g in range(CK // 16):
          di_v[pl.ds(g * 16, 16)] = dstb_v[pl.ds(j * CK + g * 16, 16)]

        def scale(j2, cy2):
          for e in range(2):
            jj = j2 * 2 + e
            w = evb_v[pl.ds(j * CK + jj, 16)][0]
            for bb in range(8):
              rows[slot, jj, pl.ds(bb * 16, 16)] = (
                  rows[slot, jj, pl.ds(bb * 16, 16)] * w)
          return cy2

        lax.fori_loop(0, CK // 2, scale, 0)
        pltpu.sync_copy(rows.at[slot], u_sh.at[di_v], add=True)
        pltpu.sync_copy(evb_v.at[pl.ds(j * CK, CK)], den_sh.at[di_v], add=True)
      return cy

    lax.fori_loop(0, CSUPC // 2, pair, 0)
    return carry

  lax.fori_loop(0, EA // CSUP, super_loop, 0)
  plsc.subcore_barrier()

  pltpu.sync_copy(den_sh.at[pl.ds(640 * s, 640)],
                  den_hbm.at[pl.ds(c * NPAD + 640 * s, 640)])

  @pl.when(s < 15)
  def _():
    pltpu.sync_copy(u_sh.at[pl.ds(640 * s, 640)],
                    out_hbm.at[pl.ds(row_off + 640 * s, 640)])

  @pl.when(s == 15)
  def _():
    pltpu.sync_copy(u_sh.at[pl.ds(9600, 400)],
                    out_hbm.at[pl.ds(row_off + 9600, 400)])


_msg = pl.kernel(
    _msg_body,
    out_type=[jax.ShapeDtypeStruct((2 * N, CH), F32),
              jax.ShapeDtypeStruct((2 * NPAD,), F32)],
    mesh=_MESH,
    scratch_types=[
        pltpu.VMEM((CSUP,), jnp.int32),
        pltpu.VMEM((CSUP,), jnp.int32),
        pltpu.VMEM((CSUP + 16,), F32),
        pltpu.VMEM((CK,), jnp.int32),
        pltpu.VMEM((2, CK, CH), F32),
        pltpu.VMEM((16,), F32),
        pltpu.VMEM_SHARED((NPAD, CH), F32),
        pltpu.VMEM_SHARED((NPAD,), F32),
        pltpu.SemaphoreType.DMA,
        pltpu.SemaphoreType.DMA,
    ],
)


_DOT = functools.partial(
    jnp.dot, preferred_element_type=F32, precision=lax.Precision.HIGHEST)


def _pre_body(x_ref, wl_ref, wr_ref, xl_ref, xr_ref):
  xb = x_ref[...]
  xl = _DOT(xb, wl_ref[...])
  xr = _DOT(xb, wr_ref[...])
  xl_ref[0] = xl[:, :CH]
  xl_ref[1] = xl[:, CH:]
  xr_ref[0] = xr[:, :CH]
  xr_ref[1] = xr[:, CH:]


_pre = pl.pallas_call(
    _pre_body,
    grid=(N // NBLK,),
    in_specs=[
        pl.BlockSpec((NBLK, 128), lambda i: (i, 0)),
        pl.BlockSpec((128, C), lambda i: (0, 0)),
        pl.BlockSpec((128, C), lambda i: (0, 0)),
    ],
    out_specs=[pl.BlockSpec((2, NBLK, CH), lambda i: (0, i, 0))] * 2,
    out_shape=[jax.ShapeDtypeStruct((2, N, CH), F32)] * 2,
)


def _unorm(u_ref, den_ref):
  u = u_ref[...]
  den = den_ref[...]
  o0 = u[0] / (den[:, 0:1] + 1e-16)
  o1 = u[1] / (den[:, 1:2] + 1e-16)
  return jnp.concatenate([o0[None], o1[None]], axis=0)


def _stats_body(u_ref, den_ref, st_ref):
  i = pl.program_id(0)
  u = _unorm(u_ref, den_ref)
  sm = jnp.sum(u, axis=1)
  sq = jnp.sum(u * u, axis=1)
  blk = jnp.concatenate(
      [sm[:, None, :], sq[:, None, :], jnp.zeros((2, 6, CH), F32)], axis=1)

  @pl.when(i == 0)
  def _():
    st_ref[...] = blk

  @pl.when(i != 0)
  def _():
    st_ref[...] = st_ref[...] + blk


_stats = pl.pallas_call(
    _stats_body,
    grid=(N // NBLK,),
    in_specs=[
        pl.BlockSpec((2, NBLK, CH), lambda i: (0, i, 0)),
        pl.BlockSpec((NBLK, 2), lambda i: (i, 0)),
    ],
    out_specs=pl.BlockSpec((2, 8, CH), lambda i: (0, 0, 0)),
    out_shape=jax.ShapeDtypeStruct((2, 8, CH), F32),
)


def _mid_body(u_ref, den_ref, st_ref, g_ref, b_ref, wl_ref, wr_ref,
              xl_ref, xr_ref):
  u = _unorm(u_ref, den_ref)
  h = jnp.concatenate([u[0], u[1]], axis=1)
  st = st_ref[...]
  sm = jnp.concatenate([st[0, 0], st[1, 0]])[None]
  sq = jnp.concatenate([st[0, 1], st[1, 1]])[None]
  mu = sm / N
  var = sq / N - mu * mu
  hn = (h - mu) * lax.rsqrt(var + 1e-5) * g_ref[...] + b_ref[...]
  hr = jnp.maximum(hn, 0.0)
  xl = _DOT(hr, wl_ref[...])
  xr = _DOT(hr, wr_ref[...])
  xl_ref[0] = xl[:, :CH]
  xl_ref[1] = xl[:, CH:]
  xr_ref[0] = xr[:, :CH]
  xr_ref[1] = xr[:, CH:]


_mid = pl.pallas_call(
    _mid_body,
    grid=(N // NBLK,),
    in_specs=[
        pl.BlockSpec((2, NBLK, CH), lambda i: (0, i, 0)),
        pl.BlockSpec((NBLK, 2), lambda i: (i, 0)),
        pl.BlockSpec((2, 8, CH), lambda i: (0, 0, 0)),
        pl.BlockSpec((1, C), lambda i: (0, 0)),
        pl.BlockSpec((1, C), lambda i: (0, 0)),
        pl.BlockSpec((C, C), lambda i: (0, 0)),
        pl.BlockSpec((C, C), lambda i: (0, 0)),
    ],
    out_specs=[pl.BlockSpec((2, NBLK, CH), lambda i: (0, i, 0))] * 2,
    out_shape=[jax.ShapeDtypeStruct((2, N, CH), F32)] * 2,
)


def _post_body(u_ref, den_ref, st_ref, b2_ref, mp_ref, dp_ref):
  u = _unorm(u_ref, den_ref)
  st = st_ref[...]
  mu = (st[0, 0] / N)[None]
  var = (st[0, 1] / N)[None] - mu * mu
  mp_ref[...] = (u[0] - mu) * lax.rsqrt(var + 1e-5)
  xd = u[1] + b2_ref[...][:, CH:]
  dp_ref[...] = jnp.maximum(xd, 0.0) + jnp.log1p(jnp.exp(-jnp.abs(xd)))


_post = pl.pallas_call(
    _post_body,
    grid=(N // NBLK,),
    in_specs=[
        pl.BlockSpec((2, NBLK, CH), lambda i: (0, i, 0)),
        pl.BlockSpec((NBLK, 2), lambda i: (i, 0)),
        pl.BlockSpec((2, 8, CH), lambda i: (0, 0, 0)),
        pl.BlockSpec((1, C), lambda i: (0, 0)),
    ],
    out_specs=[pl.BlockSpec((NBLK, CH), lambda i: (i, 0))] * 2,
    out_shape=[jax.ShapeDtypeStruct((N, CH), F32)] * 2,
)


def kernel(x, edge_index, Wl1, Wr1, att1, bias1, gamma1, beta1,
           Wl2, Wr2, att2, bias2):
  del bias1
  src = edge_index[0].astype(jnp.int32)
  dst = edge_index[1].astype(jnp.int32)
  loops = jnp.arange(N, dtype=jnp.int32)
  pad = jnp.zeros((EP - ER,), jnp.int32)
  srcp = jnp.concatenate([src, loops, pad])
  dstp = jnp.concatenate([dst, loops, pad])
  z16 = jnp.zeros((16, CH), F32)

  xl1, xr1 = _pre(x, Wl1, Wr1)
  xl1f = xl1.reshape(2 * N, CH)
  plog1 = _att(xl1f, xr1.reshape(2 * N, CH), srcp, dstp, att1.reshape(C))
  ev1 = _evred(plog1, plog1).reshape(EP)
  u1, den1 = _msg(xl1f, srcp, dstp, ev1, z16)
  u1c = u1.reshape(2, N, CH)
  den1t = den1.reshape(2, NPAD)[:, :N].T
  st1 = _stats(u1c, den1t)
  xl2, xr2 = _mid(u1c, den1t, st1, gamma1.reshape(1, C), beta1.reshape(1, C),
                  Wl2, Wr2)
  xl2f = xl2.reshape(2 * N, CH)
  plog2 = _att(xl2f, xr2.reshape(2 * N, CH), srcp, dstp, att2.reshape(C))
  ev2 = _evred(plog2, plog2).reshape(EP)
  u2, den2 = _msg(xl2f, srcp, dstp, ev2, z16)
  u2c = u2.reshape(2, N, CH)
  den2t = den2.reshape(2, NPAD)[:, :N].T
  st2 = _stats(u2c, den2t)
  mean_part, disp_part = _post(u2c, den2t, st2, bias2.reshape(1, C))
  return (mean_part, disp_part)
